# baseline (speedup 1.0000x reference)
import functools

import jax
import jax.numpy as jnp
from jax import lax
from jax.experimental import pallas as pl
from jax.experimental.pallas import tpu as pltpu
from jax.experimental.pallas import tpu_sc as plsc

VOCAB = 100000
EMBED_DIM = 300
PAD_DIM = 304
BATCH = 1024
SEQ = 200

N_IDX = BATCH * SEQ
NUM_WORKERS = 32
PER_WORKER = N_IDX // NUM_WORKERS
CHUNK = 128
NUM_CHUNKS = PER_WORKER // CHUNK

_mesh = plsc.VectorSubcoreMesh(core_axis_name="c", subcore_axis_name="s")


@functools.partial(
    pl.kernel,
    mesh=_mesh,
    out_type=jax.ShapeDtypeStruct((N_IDX, PAD_DIM), jnp.float32),
    scratch_types=[
        pltpu.VMEM((PER_WORKER,), jnp.int32),
        pltpu.VMEM((CHUNK, PAD_DIM), jnp.float32),
        pltpu.SemaphoreType.DMA,
    ],
    compiler_params=pltpu.CompilerParams(use_tc_tiling_on_sc=False),
)
def _gather_kernel(table_hbm, idx_hbm, out_hbm, idx_v, buf, sem):
    wid = lax.axis_index("s") * 2 + lax.axis_index("c")
    base = pl.multiple_of(wid * PER_WORKER, PER_WORKER)
    pltpu.sync_copy(idx_hbm.at[pl.ds(base, PER_WORKER)], idx_v)

    def body(c, carry):
        off = pl.multiple_of(c * CHUNK, CHUNK)
        idx_chunk = idx_v.at[pl.ds(off, CHUNK)]
        pltpu.async_copy(table_hbm.at[idx_chunk], buf, sem).wait()
        out_off = pl.multiple_of(base + off, CHUNK)
        pltpu.sync_copy(buf, out_hbm.at[pl.ds(out_off, CHUNK)])
        return carry

    lax.fori_loop(0, NUM_CHUNKS, body, 0)


def kernel(word_sequences, embedding):
    table = jnp.pad(embedding, ((0, 0), (0, PAD_DIM - EMBED_DIM)))
    flat_idx = word_sequences.reshape(N_IDX)
    out = _gather_kernel(table, flat_idx)
    return out[:, :EMBED_DIM].reshape(BATCH, SEQ, EMBED_DIM)
